# initial kernel scaffold (unmeasured)
import jax
import jax.numpy as jnp
from jax import lax
from jax.experimental import pallas as pl
from jax.experimental.pallas import tpu as pltpu


def kernel(
    x,
):
    def body(*refs):
        pass

    out_shape = jax.ShapeDtypeStruct(..., jnp.float32)
    return pl.pallas_call(body, out_shape=out_shape)(...)


# baseline (device time: 8368 ns/iter reference)
import jax
import jax.numpy as jnp
from jax import lax
from jax.experimental import pallas as pl
from jax.experimental.pallas import tpu as pltpu

X_SIZE = 2
Y_SIZE = 2
M_GLOBAL = 512
N_GLOBAL = 512


def kernel(x):
    m, n = x.shape

    def body(x_ref, out_ref, send_row, send_col, recv_row, recv_col,
             send_sems, recv_sems):
        my_x = lax.axis_index("x")
        my_y = lax.axis_index("y")

        barrier_sem = pltpu.get_barrier_semaphore()
        pl.semaphore_signal(barrier_sem, inc=1,
                            device_id=(1 - my_x, my_y),
                            device_id_type=pl.DeviceIdType.MESH)
        pl.semaphore_signal(barrier_sem, inc=1,
                            device_id=(my_x, 1 - my_y),
                            device_id_type=pl.DeviceIdType.MESH)
        pl.semaphore_wait(barrier_sem, 2)

        xv = x_ref[:, :]

        send_row[:, :] = jnp.where(my_x == 0, xv[m - 1:m, :], xv[0:1, :])
        send_col[:, :] = jnp.where(my_y == 0, xv[:, n - 1:n], xv[:, 0:1])

        rdma_row = pltpu.make_async_remote_copy(
            src_ref=send_row,
            dst_ref=recv_row,
            send_sem=send_sems.at[0],
            recv_sem=recv_sems.at[0],
            device_id=(1 - my_x, my_y),
            device_id_type=pl.DeviceIdType.MESH,
        )
        rdma_col = pltpu.make_async_remote_copy(
            src_ref=send_col,
            dst_ref=recv_col,
            send_sem=send_sems.at[1],
            recv_sem=recv_sems.at[1],
            device_id=(my_x, 1 - my_y),
            device_id_type=pl.DeviceIdType.MESH,
        )
        rdma_row.start()
        rdma_col.start()
        rdma_row.wait()
        rdma_col.wait()

        halo_row = recv_row[:, :]
        halo_col = recv_col[:, :]
        xN = jnp.concatenate([halo_row, xv[:-1, :]], axis=0)
        xS = jnp.concatenate([xv[1:, :], halo_row], axis=0)
        xW = jnp.concatenate([halo_col, xv[:, :-1]], axis=1)
        xE = jnp.concatenate([xv[:, 1:], halo_col], axis=1)
        stencil = 0.5 * xv + 0.125 * (xN + xS + xW + xE)

        gi = my_x * m + lax.broadcasted_iota(jnp.int32, (m, n), 0)
        gj = my_y * n + lax.broadcasted_iota(jnp.int32, (m, n), 1)
        boundary = ((gi == 0) | (gi == M_GLOBAL - 1)
                    | (gj == 0) | (gj == N_GLOBAL - 1))
        out_ref[:, :] = jnp.where(boundary, xv, stencil)

    return pl.pallas_call(
        body,
        out_shape=jax.ShapeDtypeStruct((m, n), x.dtype),
        in_specs=[pl.BlockSpec(memory_space=pltpu.VMEM)],
        out_specs=pl.BlockSpec(memory_space=pltpu.VMEM),
        scratch_shapes=[
            pltpu.VMEM((1, n), x.dtype),
            pltpu.VMEM((m, 1), x.dtype),
            pltpu.VMEM((1, n), x.dtype),
            pltpu.VMEM((m, 1), x.dtype),
            pltpu.SemaphoreType.DMA((2,)),
            pltpu.SemaphoreType.DMA((2,)),
        ],
        compiler_params=pltpu.CompilerParams(collective_id=0),
    )(x)


# device time: 8292 ns/iter; 1.0092x vs baseline; 1.0092x over previous
import jax
import jax.numpy as jnp
from jax import lax
from jax.experimental import pallas as pl
from jax.experimental.pallas import tpu as pltpu

X_SIZE = 2
Y_SIZE = 2
M_GLOBAL = 512
N_GLOBAL = 512


def kernel(x):
    m, n = x.shape

    def body(x_ref, out_ref, send_row, send_col, recv_row, recv_col,
             send_sems, recv_sems):
        my_x = lax.axis_index("x")
        my_y = lax.axis_index("y")

        xv = x_ref[:, :]

        send_row[:, :] = jnp.where(my_x == 0, xv[m - 1:m, :], xv[0:1, :])
        send_col[:, :] = jnp.where(my_y == 0, xv[:, n - 1:n], xv[:, 0:1])

        barrier_sem = pltpu.get_barrier_semaphore()
        pl.semaphore_signal(barrier_sem, inc=1,
                            device_id=(1 - my_x, my_y),
                            device_id_type=pl.DeviceIdType.MESH)
        pl.semaphore_signal(barrier_sem, inc=1,
                            device_id=(my_x, 1 - my_y),
                            device_id_type=pl.DeviceIdType.MESH)
        pl.semaphore_wait(barrier_sem, 2)

        rdma_row = pltpu.make_async_remote_copy(
            src_ref=send_row,
            dst_ref=recv_row,
            send_sem=send_sems.at[0],
            recv_sem=recv_sems.at[0],
            device_id=(1 - my_x, my_y),
            device_id_type=pl.DeviceIdType.MESH,
        )
        rdma_col = pltpu.make_async_remote_copy(
            src_ref=send_col,
            dst_ref=recv_col,
            send_sem=send_sems.at[1],
            recv_sem=recv_sems.at[1],
            device_id=(my_x, 1 - my_y),
            device_id_type=pl.DeviceIdType.MESH,
        )
        rdma_row.start()
        rdma_col.start()

        zrow = jnp.zeros((1, n), xv.dtype)
        zcol = jnp.zeros((m, 1), xv.dtype)
        xN = jnp.concatenate([zrow, xv[:-1, :]], axis=0)
        xS = jnp.concatenate([xv[1:, :], zrow], axis=0)
        xW = jnp.concatenate([zcol, xv[:, :-1]], axis=1)
        xE = jnp.concatenate([xv[:, 1:], zcol], axis=1)
        acc = 0.5 * xv + 0.125 * (xN + xS + xW + xE)

        i = lax.broadcasted_iota(jnp.int32, (m, n), 0)
        j = lax.broadcasted_iota(jnp.int32, (m, n), 1)
        row_edge = (i == 0) | (i == m - 1)
        col_edge = (j == 0) | (j == n - 1)
        gi = my_x * m + i
        gj = my_y * n + j
        boundary = ((gi == 0) | (gi == M_GLOBAL - 1)
                    | (gj == 0) | (gj == N_GLOBAL - 1))

        rdma_row.wait()
        rdma_col.wait()

        zero = jnp.zeros((m, n), xv.dtype)
        acc = acc + 0.125 * jnp.where(row_edge, recv_row[:, :], zero)
        acc = acc + 0.125 * jnp.where(col_edge, recv_col[:, :], zero)

        out_ref[:, :] = jnp.where(boundary, xv, acc)

    return pl.pallas_call(
        body,
        out_shape=jax.ShapeDtypeStruct((m, n), x.dtype),
        in_specs=[pl.BlockSpec(memory_space=pltpu.VMEM)],
        out_specs=pl.BlockSpec(memory_space=pltpu.VMEM),
        scratch_shapes=[
            pltpu.VMEM((1, n), x.dtype),
            pltpu.VMEM((m, 1), x.dtype),
            pltpu.VMEM((1, n), x.dtype),
            pltpu.VMEM((m, 1), x.dtype),
            pltpu.SemaphoreType.DMA((2,)),
            pltpu.SemaphoreType.DMA((2,)),
        ],
        compiler_params=pltpu.CompilerParams(collective_id=0),
    )(x)
